# baseline (speedup 1.0000x reference)
import functools

import jax
import jax.numpy as jnp
from jax.experimental import pallas as pl
from jax.experimental.pallas import tpu as pltpu
from jax.experimental.pallas import tpu_sc as plsc

D_MODEL = 1024
D_FF = 4096
E = 8
TOP_K = 2
CAP = 1280

N_TOK = 4096
TBLK = 512
N_TB = N_TOK // TBLK
CTILE = 256
N_CT = CAP // CTILE
S_TOT = E * CAP
DBLK = 256


def _router_kernel(x_ref, g_ref, b_ref, wg_ref,
                   xln_ref, table_ref, sid1_ref, sid2_ref, we1_ref, we2_ref,
                   carry_ref):
    t = pl.program_id(0)

    @pl.when(t == 0)
    def _init():
        carry_ref[...] = jnp.zeros_like(carry_ref)
        table_ref[...] = jnp.zeros_like(table_ref)

    x = x_ref[...]
    mu = jnp.mean(x, axis=1, keepdims=True)
    xc = x - mu
    var = jnp.mean(xc * xc, axis=1, keepdims=True)
    xln = xc * jax.lax.rsqrt(var + 1e-5) * g_ref[...] + b_ref[...]
    xln_ref[...] = xln

    logits = jnp.dot(xln, wg_ref[...], preferred_element_type=jnp.float32)
    col = jax.lax.broadcasted_iota(jnp.int32, (TBLK, 128), 1)
    neg = jnp.float32(-1e30)
    logits = jnp.where(col < E, logits, neg)
    v1 = jnp.max(logits, axis=1, keepdims=True)
    i1 = jnp.min(jnp.where(logits == v1, col, 128), axis=1, keepdims=True)
    l2 = jnp.where(col == i1, neg, logits)
    v2 = jnp.max(l2, axis=1, keepdims=True)
    i2 = jnp.min(jnp.where(l2 == v2, col, 128), axis=1, keepdims=True)
    w1 = 1.0 / (1.0 + jnp.exp(v2 - v1))
    w2 = 1.0 - w1

    oh1 = (col == i1).astype(jnp.float32)
    oh2 = (col == i2).astype(jnp.float32)
    ohsum = oh1 + oh2
    ri = jax.lax.broadcasted_iota(jnp.int32, (TBLK, TBLK), 0)
    ci = jax.lax.broadcasted_iota(jnp.int32, (TBLK, TBLK), 1)
    tri = (ci < ri).astype(jnp.float32)
    s_excl = jnp.dot(tri, ohsum, preferred_element_type=jnp.float32,
                     precision=jax.lax.Precision.HIGHEST)
    base = carry_ref[...] + s_excl
    pos1 = jnp.round(jnp.sum(oh1 * base, axis=1, keepdims=True))
    pos2 = jnp.round(jnp.sum(oh2 * (base + oh1), axis=1, keepdims=True))
    carry_ref[...] += jnp.sum(ohsum, axis=0, keepdims=True)

    keep1 = (pos1 < CAP).astype(jnp.float32)
    keep2 = (pos2 < CAP).astype(jnp.float32)
    pos1c = jnp.minimum(pos1, CAP - 1)
    pos2c = jnp.minimum(pos2, CAP - 1)

    e1f = jnp.sum(oh1 * col.astype(jnp.float32), axis=1, keepdims=True)
    e2f = jnp.sum(oh2 * col.astype(jnp.float32), axis=1, keepdims=True)
    sid1_ref[...] = (e1f * CAP + pos1c).astype(jnp.int32)
    sid2_ref[...] = (e2f * CAP + pos2c).astype(jnp.int32)
    we1_ref[...] = w1 * keep1
    we2_ref[...] = w2 * keep2

    gtok = (jax.lax.broadcasted_iota(jnp.int32, (TBLK, 1), 0)
            + t * TBLK).astype(jnp.float32)
    cdiv3 = col // 3
    csel = col - cdiv3 * 3
    valid_col = col < 3 * E

    def make_vals(i_e, w_eff, tokpay):
        eq = (cdiv3 == i_e) & valid_col
        pay = jnp.where(csel == 0, tokpay,
                        jnp.where(csel == 1, w_eff, (w_eff != 0).astype(jnp.float32)))
        return jnp.where(eq, pay, 0.0)

    vals1 = make_vals(i1, w1 * keep1, (gtok + 1.0) * keep1)
    vals2 = make_vals(i2, w2 * keep2, (gtok + 1.0) * keep2)

    pr = jax.lax.broadcasted_iota(jnp.int32, (TBLK, CAP), 1)
    p1 = ((pr == pos1c.astype(jnp.int32)) & (keep1 > 0)).astype(jnp.float32)
    p2 = ((pr == pos2c.astype(jnp.int32)) & (keep2 > 0)).astype(jnp.float32)
    dnum = (((0,), (0,)), ((), ()))
    hi = jax.lax.Precision.HIGHEST
    table_ref[...] += (
        jax.lax.dot_general(p1, vals1, dnum, preferred_element_type=jnp.float32,
                            precision=hi)
        + jax.lax.dot_general(p2, vals2, dnum, preferred_element_type=jnp.float32,
                              precision=hi))


def _router_call(x2d, gamma, beta, wg_pad):
    return pl.pallas_call(
        _router_kernel,
        grid=(N_TB,),
        in_specs=[
            pl.BlockSpec((TBLK, D_MODEL), lambda t: (t, 0)),
            pl.BlockSpec((1, D_MODEL), lambda t: (0, 0)),
            pl.BlockSpec((1, D_MODEL), lambda t: (0, 0)),
            pl.BlockSpec((D_MODEL, 128), lambda t: (0, 0)),
        ],
        out_specs=[
            pl.BlockSpec((TBLK, D_MODEL), lambda t: (t, 0)),
            pl.BlockSpec((CAP, 128), lambda t: (0, 0)),
            pl.BlockSpec((TBLK, 1), lambda t: (t, 0)),
            pl.BlockSpec((TBLK, 1), lambda t: (t, 0)),
            pl.BlockSpec((TBLK, 1), lambda t: (t, 0)),
            pl.BlockSpec((TBLK, 1), lambda t: (t, 0)),
        ],
        out_shape=[
            jax.ShapeDtypeStruct((N_TOK, D_MODEL), jnp.float32),
            jax.ShapeDtypeStruct((CAP, 128), jnp.float32),
            jax.ShapeDtypeStruct((N_TOK, 1), jnp.int32),
            jax.ShapeDtypeStruct((N_TOK, 1), jnp.int32),
            jax.ShapeDtypeStruct((N_TOK, 1), jnp.float32),
            jax.ShapeDtypeStruct((N_TOK, 1), jnp.float32),
        ],
        scratch_shapes=[pltpu.VMEM((1, 128), jnp.float32)],
        compiler_params=pltpu.CompilerParams(
            dimension_semantics=("arbitrary",)),
    )(x2d, gamma, beta, wg_pad)


_NW = 32
_CH = 32


def _sc_gather_call(table, idx, nrows):
    b_per_w = nrows // _NW
    nch = b_per_w // _CH
    mesh = plsc.VectorSubcoreMesh(core_axis_name="c", subcore_axis_name="s")

    @functools.partial(
        pl.kernel, mesh=mesh,
        out_type=jax.ShapeDtypeStruct((nrows, D_MODEL), jnp.float32),
        scratch_types=[
            pltpu.VMEM((_CH,), jnp.int32),
            pltpu.VMEM((_CH, D_MODEL), jnp.float32),
            pltpu.SemaphoreType.DMA,
        ],
    )
    def k(table_hbm, idx_hbm, out_hbm, idx_v, rows_v, sem):
        wid = jax.lax.axis_index("s") * 2 + jax.lax.axis_index("c")
        base = wid * b_per_w

        def body(i, carry):
            off = base + i * _CH
            pltpu.sync_copy(idx_hbm.at[pl.ds(off, _CH)], idx_v)
            pltpu.async_copy(table_hbm.at[idx_v], rows_v, sem).wait()
            pltpu.sync_copy(rows_v, out_hbm.at[pl.ds(off, _CH)])
            return carry

        jax.lax.fori_loop(0, nch, body, 0)

    return k(table, idx)


def _ffn_kernel(buf_ref, w1_ref, b1_ref, w2_ref, b2_ref, out_ref):
    x = buf_ref[...].astype(jnp.bfloat16)
    h = jnp.dot(x, w1_ref[0], preferred_element_type=jnp.float32) + b1_ref[0]
    h = jax.nn.gelu(h)
    o = jnp.dot(h.astype(jnp.bfloat16), w2_ref[0],
                preferred_element_type=jnp.float32) + b2_ref[0]
    out_ref[...] = o


def _ffn_call(buf, w1, b1, w2, b2):
    return pl.pallas_call(
        _ffn_kernel,
        grid=(E, N_CT),
        in_specs=[
            pl.BlockSpec((CTILE, D_MODEL), lambda e, t: (e * N_CT + t, 0)),
            pl.BlockSpec((1, D_MODEL, D_FF), lambda e, t: (e, 0, 0)),
            pl.BlockSpec((1, 1, D_FF), lambda e, t: (e, 0, 0)),
            pl.BlockSpec((1, D_FF, D_MODEL), lambda e, t: (e, 0, 0)),
            pl.BlockSpec((1, 1, D_MODEL), lambda e, t: (e, 0, 0)),
        ],
        out_specs=pl.BlockSpec((CTILE, D_MODEL), lambda e, t: (e * N_CT + t, 0)),
        out_shape=jax.ShapeDtypeStruct((S_TOT, D_MODEL), jnp.float32),
        compiler_params=pltpu.CompilerParams(
            dimension_semantics=("arbitrary", "arbitrary")),
    )(buf, w1, b1, w2, b2)


def _final_kernel(y0_ref, y1_ref, we1_ref, we2_ref, data_ref, y_ref):
    y_ref[...] = (data_ref[...] + we1_ref[...] * y0_ref[...]
                  + we2_ref[...] * y1_ref[...])


def _final_call(y0, y1, we1, we2, data2d):
    nb = N_TOK // DBLK
    return pl.pallas_call(
        _final_kernel,
        grid=(nb,),
        in_specs=[
            pl.BlockSpec((DBLK, D_MODEL), lambda t: (t, 0)),
            pl.BlockSpec((DBLK, D_MODEL), lambda t: (t, 0)),
            pl.BlockSpec((DBLK, 1), lambda t: (t, 0)),
            pl.BlockSpec((DBLK, 1), lambda t: (t, 0)),
            pl.BlockSpec((DBLK, D_MODEL), lambda t: (t, 0)),
        ],
        out_specs=pl.BlockSpec((DBLK, D_MODEL), lambda t: (t, 0)),
        out_shape=jax.ShapeDtypeStruct((N_TOK, D_MODEL), jnp.float32),
        compiler_params=pltpu.CompilerParams(
            dimension_semantics=("arbitrary",)),
    )(y0, y1, we1, we2, data2d)


@jax.jit
def kernel(data, gamma, beta, Wg, W1, b1, W2, b2):
    B, S, D = data.shape
    x2d = data.reshape(B * S, D)
    wg_pad = jnp.zeros((D_MODEL, 128), jnp.float32).at[:, :E].set(Wg)
    g2 = gamma.reshape(1, D_MODEL)
    b2d = beta.reshape(1, D_MODEL)

    xln, table, sid1, sid2, we1, we2 = _router_call(x2d, g2, b2d, wg_pad)

    s_tok = jnp.maximum(
        jnp.round(table[:, 0:3 * E:3].T.reshape(S_TOT)).astype(jnp.int32) - 1, 0)

    buf = _sc_gather_call(xln, s_tok, S_TOT)

    w1bf = W1.astype(jnp.bfloat16)
    w2bf = W2.astype(jnp.bfloat16)
    out_e = _ffn_call(buf, w1bf, b1.reshape(E, 1, D_FF),
                      w2bf, b2.reshape(E, 1, D_MODEL))

    sidcat = jnp.concatenate([sid1.reshape(N_TOK), sid2.reshape(N_TOK)])
    y_rep = _sc_gather_call(out_e, sidcat, 2 * N_TOK)

    y = _final_call(y_rep[:N_TOK], y_rep[N_TOK:], we1, we2, x2d)
    return y.reshape(B, S, D)


# speedup vs baseline: 1.2094x; 1.2094x over previous
import functools

import jax
import jax.numpy as jnp
from jax.experimental import pallas as pl
from jax.experimental.pallas import tpu as pltpu
from jax.experimental.pallas import tpu_sc as plsc

D_MODEL = 1024
D_FF = 4096
E = 8
TOP_K = 2
CAP = 1280

N_TOK = 4096
TBLK = 512
N_TB = N_TOK // TBLK
CTILE = 256
N_CT = CAP // CTILE
S_TOT = E * CAP
DBLK = 256


def _router_kernel(x_ref, g_ref, b_ref, wg_ref,
                   xln_ref, table_ref, sid1_ref, sid2_ref, we1_ref, we2_ref,
                   carry_ref):
    t = pl.program_id(0)

    @pl.when(t == 0)
    def _init():
        carry_ref[...] = jnp.zeros_like(carry_ref)
        table_ref[...] = jnp.zeros_like(table_ref)

    x = x_ref[...]
    mu = jnp.mean(x, axis=1, keepdims=True)
    xc = x - mu
    var = jnp.mean(xc * xc, axis=1, keepdims=True)
    xln = xc * jax.lax.rsqrt(var + 1e-5) * g_ref[...] + b_ref[...]
    xln_ref[...] = xln.astype(jnp.bfloat16)

    logits = jnp.dot(xln, wg_ref[...], preferred_element_type=jnp.float32)
    col = jax.lax.broadcasted_iota(jnp.int32, (TBLK, 128), 1)
    neg = jnp.float32(-1e30)
    logits = jnp.where(col < E, logits, neg)
    v1 = jnp.max(logits, axis=1, keepdims=True)
    i1 = jnp.min(jnp.where(logits == v1, col, 128), axis=1, keepdims=True)
    l2 = jnp.where(col == i1, neg, logits)
    v2 = jnp.max(l2, axis=1, keepdims=True)
    i2 = jnp.min(jnp.where(l2 == v2, col, 128), axis=1, keepdims=True)
    w1 = 1.0 / (1.0 + jnp.exp(v2 - v1))
    w2 = 1.0 - w1

    oh1 = (col == i1).astype(jnp.float32)
    oh2 = (col == i2).astype(jnp.float32)
    ohsum = oh1 + oh2
    ri = jax.lax.broadcasted_iota(jnp.int32, (TBLK, TBLK), 0)
    ci = jax.lax.broadcasted_iota(jnp.int32, (TBLK, TBLK), 1)
    tri = (ci < ri).astype(jnp.float32)
    s_excl = jnp.dot(tri, ohsum, preferred_element_type=jnp.float32,
                     precision=jax.lax.Precision.HIGHEST)
    base = carry_ref[...] + s_excl
    pos1 = jnp.round(jnp.sum(oh1 * base, axis=1, keepdims=True))
    pos2 = jnp.round(jnp.sum(oh2 * (base + oh1), axis=1, keepdims=True))
    carry_ref[...] += jnp.sum(ohsum, axis=0, keepdims=True)

    keep1 = (pos1 < CAP).astype(jnp.float32)
    keep2 = (pos2 < CAP).astype(jnp.float32)
    pos1c = jnp.minimum(pos1, CAP - 1)
    pos2c = jnp.minimum(pos2, CAP - 1)

    e1f = jnp.sum(oh1 * col.astype(jnp.float32), axis=1, keepdims=True)
    e2f = jnp.sum(oh2 * col.astype(jnp.float32), axis=1, keepdims=True)
    sid1_ref[...] = (e1f * CAP + pos1c).astype(jnp.int32)
    sid2_ref[...] = (e2f * CAP + pos2c).astype(jnp.int32)
    we1_ref[...] = w1 * keep1
    we2_ref[...] = w2 * keep2

    gtok = (jax.lax.broadcasted_iota(jnp.int32, (TBLK, 1), 0)
            + t * TBLK).astype(jnp.float32)
    cdiv3 = col // 3
    csel = col - cdiv3 * 3
    valid_col = col < 3 * E

    def make_vals(i_e, w_eff, tokpay):
        eq = (cdiv3 == i_e) & valid_col
        pay = jnp.where(csel == 0, tokpay,
                        jnp.where(csel == 1, w_eff, (w_eff != 0).astype(jnp.float32)))
        return jnp.where(eq, pay, 0.0)

    vals1 = make_vals(i1, w1 * keep1, (gtok + 1.0) * keep1)
    vals2 = make_vals(i2, w2 * keep2, (gtok + 1.0) * keep2)

    pr = jax.lax.broadcasted_iota(jnp.int32, (TBLK, CAP), 1)
    p1 = ((pr == pos1c.astype(jnp.int32)) & (keep1 > 0)).astype(jnp.float32)
    p2 = ((pr == pos2c.astype(jnp.int32)) & (keep2 > 0)).astype(jnp.float32)
    dnum = (((0,), (0,)), ((), ()))
    hi = jax.lax.Precision.HIGHEST
    table_ref[...] += (
        jax.lax.dot_general(p1, vals1, dnum, preferred_element_type=jnp.float32,
                            precision=hi)
        + jax.lax.dot_general(p2, vals2, dnum, preferred_element_type=jnp.float32,
                              precision=hi))


def _router_call(x2d, gamma, beta, wg_pad):
    return pl.pallas_call(
        _router_kernel,
        grid=(N_TB,),
        in_specs=[
            pl.BlockSpec((TBLK, D_MODEL), lambda t: (t, 0)),
            pl.BlockSpec((1, D_MODEL), lambda t: (0, 0)),
            pl.BlockSpec((1, D_MODEL), lambda t: (0, 0)),
            pl.BlockSpec((D_MODEL, 128), lambda t: (0, 0)),
        ],
        out_specs=[
            pl.BlockSpec((TBLK, D_MODEL), lambda t: (t, 0)),
            pl.BlockSpec((CAP, 128), lambda t: (0, 0)),
            pl.BlockSpec((TBLK, 1), lambda t: (t, 0)),
            pl.BlockSpec((TBLK, 1), lambda t: (t, 0)),
            pl.BlockSpec((TBLK, 1), lambda t: (t, 0)),
            pl.BlockSpec((TBLK, 1), lambda t: (t, 0)),
        ],
        out_shape=[
            jax.ShapeDtypeStruct((N_TOK, D_MODEL), jnp.bfloat16),
            jax.ShapeDtypeStruct((CAP, 128), jnp.float32),
            jax.ShapeDtypeStruct((N_TOK, 1), jnp.int32),
            jax.ShapeDtypeStruct((N_TOK, 1), jnp.int32),
            jax.ShapeDtypeStruct((N_TOK, 1), jnp.float32),
            jax.ShapeDtypeStruct((N_TOK, 1), jnp.float32),
        ],
        scratch_shapes=[pltpu.VMEM((1, 128), jnp.float32)],
        compiler_params=pltpu.CompilerParams(
            dimension_semantics=("arbitrary",)),
    )(x2d, gamma, beta, wg_pad)


F_SPLIT = 2
D_FH = D_FF // F_SPLIT
NB_S = S_TOT // CTILE


def _ffn_kernel(stok_ref, xln_ref, w1_ref, b1_ref, w2_ref, b2_ref, out_ref,
                xe_ref, oacc_ref):
    f = pl.program_id(1)
    t = pl.program_id(2)
    row0 = t * CTILE

    @pl.when(f == 0)
    def _gather():
        tok = stok_ref[...]
        it = jax.lax.broadcasted_iota(jnp.int32, (CTILE, N_TOK), 1)
        oh = (it == tok).astype(jnp.bfloat16)
        x = jnp.dot(oh, xln_ref[...], preferred_element_type=jnp.float32)
        xe_ref[pl.ds(row0, CTILE), :] = x

    x = xe_ref[pl.ds(row0, CTILE), :].astype(jnp.bfloat16)
    h = jnp.dot(x, w1_ref[0].astype(jnp.bfloat16),
                preferred_element_type=jnp.float32) + b1_ref[0]
    h = jax.nn.gelu(h)
    p = jnp.dot(h.astype(jnp.bfloat16), w2_ref[0].astype(jnp.bfloat16),
                preferred_element_type=jnp.float32)

    @pl.when(f == 0)
    def _first():
        oacc_ref[pl.ds(row0, CTILE), :] = p + b2_ref[0]
        out_ref[...] = p.astype(jnp.bfloat16)

    @pl.when(f == F_SPLIT - 1)
    def _last():
        out_ref[...] = (oacc_ref[pl.ds(row0, CTILE), :] + p).astype(jnp.bfloat16)


def _ffn_call(s_tok, xln, w1, b1, w2, b2):
    return pl.pallas_call(
        _ffn_kernel,
        grid=(E, F_SPLIT, N_CT),
        in_specs=[
            pl.BlockSpec((CTILE, 1), lambda e, f, t: (e * N_CT + t, 0)),
            pl.BlockSpec((N_TOK, D_MODEL), lambda e, f, t: (0, 0)),
            pl.BlockSpec((1, D_MODEL, D_FH), lambda e, f, t: (e, 0, f)),
            pl.BlockSpec((1, 1, D_FH), lambda e, f, t: (e, 0, f)),
            pl.BlockSpec((1, D_FH, D_MODEL), lambda e, f, t: (e, f, 0)),
            pl.BlockSpec((1, 1, D_MODEL), lambda e, f, t: (e, 0, 0)),
        ],
        out_specs=pl.BlockSpec(
            (CTILE, D_MODEL), lambda e, f, t: (f * NB_S + e * N_CT + t, 0)),
        out_shape=jax.ShapeDtypeStruct((F_SPLIT * S_TOT, D_MODEL), jnp.bfloat16),
        scratch_shapes=[
            pltpu.VMEM((CAP, D_MODEL), jnp.float32),
            pltpu.VMEM((CAP, D_MODEL), jnp.float32),
        ],
        compiler_params=pltpu.CompilerParams(
            dimension_semantics=("arbitrary", "arbitrary", "arbitrary")),
    )(s_tok, xln, w1, b1, w2, b2)


def _combine_kernel(sid1_ref, sid2_ref, we1_ref, we2_ref, oute_ref, data_ref,
                    y_ref):
    si = jax.lax.broadcasted_iota(jnp.int32, (DBLK, S_TOT), 1)
    oh = (jnp.where(si == sid1_ref[...], we1_ref[...], 0.0)
          + jnp.where(si == sid2_ref[...], we2_ref[...], 0.0)).astype(jnp.bfloat16)
    y = jnp.dot(oh, oute_ref[...], preferred_element_type=jnp.float32)
    y_ref[...] = y + data_ref[...]


def _combine_call(sid1, sid2, we1, we2, out_e, data2d):
    nb = N_TOK // DBLK
    return pl.pallas_call(
        _combine_kernel,
        grid=(nb,),
        in_specs=[
            pl.BlockSpec((DBLK, 1), lambda t: (t, 0)),
            pl.BlockSpec((DBLK, 1), lambda t: (t, 0)),
            pl.BlockSpec((DBLK, 1), lambda t: (t, 0)),
            pl.BlockSpec((DBLK, 1), lambda t: (t, 0)),
            pl.BlockSpec((S_TOT, D_MODEL), lambda t: (F_SPLIT - 1, 0)),
            pl.BlockSpec((DBLK, D_MODEL), lambda t: (t, 0)),
        ],
        out_specs=pl.BlockSpec((DBLK, D_MODEL), lambda t: (t, 0)),
        out_shape=jax.ShapeDtypeStruct((N_TOK, D_MODEL), jnp.float32),
        compiler_params=pltpu.CompilerParams(
            dimension_semantics=("arbitrary",)),
    )(sid1, sid2, we1, we2, out_e, data2d)


@jax.jit
def kernel(data, gamma, beta, Wg, W1, b1, W2, b2):
    B, S, D = data.shape
    x2d = data.reshape(B * S, D)
    wg_pad = jnp.zeros((D_MODEL, 128), jnp.float32).at[:, :E].set(Wg)
    g2 = gamma.reshape(1, D_MODEL)
    b2d = beta.reshape(1, D_MODEL)

    xln, table, sid1, sid2, we1, we2 = _router_call(x2d, g2, b2d, wg_pad)

    s_tok = jnp.round(table[:, 0:3 * E:3].T.reshape(S_TOT, 1)).astype(jnp.int32) - 1

    out_e = _ffn_call(s_tok, xln, W1, b1.reshape(E, 1, D_FF),
                      W2, b2.reshape(E, 1, D_MODEL))

    y = _combine_call(sid1, sid2, we1, we2, out_e, x2d)
    return y.reshape(B, S, D)
